# baseline (speedup 1.0000x reference)
import functools

import jax
import jax.numpy as jnp
from jax import lax
from jax.experimental import pallas as pl
from jax.experimental.pallas import tpu as pltpu
from jax.experimental.pallas import tpu_sc as plsc

_N_COMPONENTS = 1024
_EMBEDDING_DIM = 64
_BETA = 0.25
_BLK = 2304
_ROWS = 36864

_info = plsc.get_sparse_core_info()
_NW = _info.num_cores * _info.num_subcores
_B_PER_W = _ROWS // _NW


def _vq_block(x_ref, cb_ref, soft_ref, idx_ref, loss_ref, c2_ref):
    @pl.when(pl.program_id(0) == 0)
    def _prologue():
        cb0 = cb_ref[...]
        c2_ref[...] = jnp.sum(cb0 * cb0, axis=0, keepdims=True)
        loss_ref[...] = jnp.zeros_like(loss_ref)

    x = x_ref[...]
    cb = cb_ref[...]
    sim = jnp.dot(x, cb, preferred_element_type=jnp.float32)
    x2 = jnp.sum(x * x, axis=1, keepdims=True)
    dist = x2 + c2_ref[...] - 2.0 * sim
    s = (1.0 / dist) ** 2
    soft_ref[...] = s / jnp.sum(s, axis=1, keepdims=True)
    idx_ref[...] = jnp.argmin(dist, axis=1).reshape(1, 1, _BLK)
    mind = jnp.min(dist, axis=1)
    loss_ref[...] += jnp.sum(mind).reshape(1, 1)


def _tc_part(flat, codebook):
    grid = _ROWS // _BLK
    return pl.pallas_call(
        _vq_block,
        grid=(grid,),
        in_specs=[
            pl.BlockSpec((_BLK, _EMBEDDING_DIM), lambda i: (i, 0)),
            pl.BlockSpec((_EMBEDDING_DIM, _N_COMPONENTS), lambda i: (0, 0)),
        ],
        out_specs=[
            pl.BlockSpec((_BLK, _N_COMPONENTS), lambda i: (i, 0)),
            pl.BlockSpec((1, 1, _BLK), lambda i: (i, 0, 0)),
            pl.BlockSpec((1, 1), lambda i: (0, 0)),
        ],
        out_shape=[
            jax.ShapeDtypeStruct((_ROWS, _N_COMPONENTS), jnp.float32),
            jax.ShapeDtypeStruct((_ROWS // _BLK, 1, _BLK), jnp.int32),
            jax.ShapeDtypeStruct((1, 1), jnp.float32),
        ],
        scratch_shapes=[pltpu.VMEM((1, _N_COMPONENTS), jnp.float32)],
    )(flat, codebook)


_CHUNK = 128
_K = _B_PER_W // _CHUNK


@functools.partial(
    pl.kernel,
    mesh=plsc.VectorSubcoreMesh(core_axis_name="c", subcore_axis_name="s"),
    out_type=jax.ShapeDtypeStruct((_ROWS, _EMBEDDING_DIM), jnp.float32),
    scratch_types=[
        pltpu.VMEM((_K, _CHUNK), jnp.int32),
        pltpu.VMEM((_B_PER_W, _EMBEDDING_DIM), jnp.float32),
        pltpu.SemaphoreType.DMA,
    ],
    compiler_params=pltpu.CompilerParams(use_tc_tiling_on_sc=False),
)
def _sc_gather(table_hbm, idx_hbm, out_hbm, idx_v, rows_v, sem):
    wid = lax.axis_index("s") * _info.num_cores + lax.axis_index("c")
    base = wid * _B_PER_W
    pltpu.sync_copy(idx_hbm.at[wid], idx_v)
    copies = [
        pltpu.async_copy(table_hbm.at[idx_v.at[j]],
                         rows_v.at[pl.ds(j * _CHUNK, _CHUNK)], sem)
        for j in range(_K)
    ]
    for c in copies:
        c.wait()
    pltpu.sync_copy(rows_v, out_hbm.at[pl.ds(base, _B_PER_W)])


def kernel(x, codebook):
    input_shape = x.shape
    flat = x.reshape(-1, _EMBEDDING_DIM)
    soft, idx3, loss = _tc_part(flat, codebook)
    table = codebook.T.reshape(_N_COMPONENTS, _EMBEDDING_DIM)
    q = _sc_gather(table, idx3.reshape(_NW, _K, _CHUNK))
    quantized = q.reshape(input_shape)
    vq_loss = (1.0 + _BETA) * loss[0, 0] / flat.size
    return quantized, soft, vq_loss


# speedup vs baseline: 1.1672x; 1.1672x over previous
import jax
import jax.numpy as jnp
from jax.experimental import pallas as pl
from jax.experimental.pallas import tpu as pltpu

_N_COMPONENTS = 1024
_EMBEDDING_DIM = 64
_BETA = 0.25
_BLK = 3072


def _vq_block(x_ref, cb_ref, soft_ref, q_ref, loss_ref, c2_ref):
    @pl.when(pl.program_id(0) == 0)
    def _prologue():
        cb0 = cb_ref[...]
        c2_ref[...] = jnp.sum(cb0 * cb0, axis=0, keepdims=True)
        loss_ref[...] = jnp.zeros_like(loss_ref)

    x = x_ref[...]
    cb = cb_ref[...]
    sim = jnp.dot(x, cb, preferred_element_type=jnp.float32)
    x2 = jnp.sum(x * x, axis=1, keepdims=True)
    dist = x2 + c2_ref[...] - 2.0 * sim
    s = (1.0 / dist) ** 2
    soft_ref[...] = s / jnp.sum(s, axis=1, keepdims=True)
    idx = jnp.argmin(dist, axis=1)
    enc = (jax.lax.broadcasted_iota(jnp.int32, (_BLK, _N_COMPONENTS), 1)
           == idx[:, None]).astype(jnp.float32)
    q = jax.lax.dot_general(enc, cb,
                            dimension_numbers=(((1,), (1,)), ((), ())),
                            preferred_element_type=jnp.float32)
    q_ref[...] = q
    mind = jnp.min(dist, axis=1)
    loss_ref[...] += jnp.sum(mind).reshape(1, 1)


def kernel(x, codebook):
    input_shape = x.shape
    flat = x.reshape(-1, _EMBEDDING_DIM)
    rows = flat.shape[0]
    grid = rows // _BLK

    soft, q, loss = pl.pallas_call(
        _vq_block,
        grid=(grid,),
        in_specs=[
            pl.BlockSpec((_BLK, _EMBEDDING_DIM), lambda i: (i, 0)),
            pl.BlockSpec((_EMBEDDING_DIM, _N_COMPONENTS), lambda i: (0, 0)),
        ],
        out_specs=[
            pl.BlockSpec((_BLK, _N_COMPONENTS), lambda i: (i, 0)),
            pl.BlockSpec((_BLK, _EMBEDDING_DIM), lambda i: (i, 0)),
            pl.BlockSpec((1, 1), lambda i: (0, 0)),
        ],
        out_shape=[
            jax.ShapeDtypeStruct((rows, _N_COMPONENTS), jnp.float32),
            jax.ShapeDtypeStruct((rows, _EMBEDDING_DIM), jnp.float32),
            jax.ShapeDtypeStruct((1, 1), jnp.float32),
        ],
        scratch_shapes=[pltpu.VMEM((1, _N_COMPONENTS), jnp.float32)],
    )(flat, codebook)

    quantized = q.reshape(input_shape)
    vq_loss = (1.0 + _BETA) * loss[0, 0] / flat.size
    return quantized, soft, vq_loss
